# baseline (speedup 1.0000x reference)
import functools

import jax
import jax.numpy as jnp
from jax import lax
from jax.experimental import pallas as pl
from jax.experimental.pallas import tpu as pltpu
from jax.experimental.pallas import tpu_sc as plsc

MIN_R = 1.0
MAX_R = 5.0

_NC = 2
_NS = 16
_NW = _NC * _NS
_CHUNK = 128


def _sc_gather(user_table, item_table, u_idx3, i_idx3, B, D):
    n_chunks = u_idx3.shape[1]
    b_per_w = n_chunks * _CHUNK
    mesh = plsc.VectorSubcoreMesh(core_axis_name="c", subcore_axis_name="s")

    @functools.partial(
        pl.kernel,
        out_type=(
            jax.ShapeDtypeStruct((B, D), jnp.float32),
            jax.ShapeDtypeStruct((B, D), jnp.float32),
        ),
        mesh=mesh,
        compiler_params=pltpu.CompilerParams(use_tc_tiling_on_sc=False),
        scratch_types=[
            pltpu.VMEM((n_chunks, _CHUNK), jnp.int32),
            pltpu.VMEM((n_chunks, _CHUNK), jnp.int32),
            pltpu.VMEM((b_per_w, D), jnp.float32),
            pltpu.VMEM((b_per_w, D), jnp.float32),
            pltpu.SemaphoreType.DMA,
        ],
    )
    def gather_kernel(ut_hbm, it_hbm, uidx_hbm, iidx_hbm, uout_hbm, iout_hbm,
                      uidx_v, iidx_v, urows_v, irows_v, sem):
        wid = lax.axis_index("s") * _NC + lax.axis_index("c")
        base = wid * b_per_w
        pltpu.sync_copy(uidx_hbm.at[wid], uidx_v)
        pltpu.sync_copy(iidx_hbm.at[wid], iidx_v)
        copies = []
        for j in range(n_chunks):
            copies.append(pltpu.async_copy(
                ut_hbm.at[uidx_v.at[j]], urows_v.at[pl.ds(j * _CHUNK, _CHUNK)], sem))
            copies.append(pltpu.async_copy(
                it_hbm.at[iidx_v.at[j]], irows_v.at[pl.ds(j * _CHUNK, _CHUNK)], sem))
        for c in copies:
            c.wait()
        pltpu.sync_copy(urows_v, uout_hbm.at[pl.ds(base, b_per_w)])
        pltpu.sync_copy(irows_v, iout_hbm.at[pl.ds(base, b_per_w)])

    return gather_kernel(user_table, item_table, u_idx3, i_idx3)


def _mlp_body(u_ref, i_ref, w0_ref, b0_ref, w1_ref, b1_ref, w2_ref, b2_ref,
              w3_ref, b3_ref, out_ref):
    u = u_ref[...]
    i = i_ref[...]
    w0 = w0_ref[...]
    h = jnp.dot(u, w0[0:64], preferred_element_type=jnp.float32)
    h = h + jnp.dot(i, w0[64:128], preferred_element_type=jnp.float32)
    h = jax.nn.relu(h + b0_ref[...])
    h = jax.nn.relu(jnp.dot(h, w1_ref[...], preferred_element_type=jnp.float32)
                    + b1_ref[...])
    h = jax.nn.relu(jnp.dot(h, w2_ref[...], preferred_element_type=jnp.float32)
                    + b2_ref[...])
    r = jnp.sum(h * w3_ref[...], axis=1) + b3_ref[0, 0]
    out_ref[...] = jnp.clip(r, MIN_R, MAX_R)


def _tc_mlp(U_emb, I_emb, W0, b0, W1, b1, W2, b2, W3, b3):
    B, D = U_emb.shape
    TB = 2048
    grid = (B // TB,)
    b0r = b0.reshape(1, D)
    b1r = b1.reshape(1, D)
    b2r = b2.reshape(1, D)
    w3r = W3.reshape(1, D)
    b3r = b3.reshape(1, 1)
    return pl.pallas_call(
        _mlp_body,
        grid=grid,
        in_specs=[
            pl.BlockSpec((TB, D), lambda i: (i, 0)),
            pl.BlockSpec((TB, D), lambda i: (i, 0)),
            pl.BlockSpec((2 * D, D), lambda i: (0, 0)),
            pl.BlockSpec((1, D), lambda i: (0, 0)),
            pl.BlockSpec((D, D), lambda i: (0, 0)),
            pl.BlockSpec((1, D), lambda i: (0, 0)),
            pl.BlockSpec((D, D), lambda i: (0, 0)),
            pl.BlockSpec((1, D), lambda i: (0, 0)),
            pl.BlockSpec((1, D), lambda i: (0, 0)),
            pl.BlockSpec((1, 1), lambda i: (0, 0)),
        ],
        out_specs=pl.BlockSpec((TB,), lambda i: (i,)),
        out_shape=jax.ShapeDtypeStruct((B,), jnp.float32),
    )(U_emb, I_emb, W0, b0r, W1, b1r, W2, b2r, w3r, b3r)


def kernel(U_ids, I_ids, user_table, item_table, W0, b0, W1, b1, W2, b2, W3, b3):
    B = U_ids.shape[0]
    D = user_table.shape[1]
    b_per_w = B // _NW
    n_chunks = b_per_w // _CHUNK
    u_idx3 = U_ids.reshape(_NW, n_chunks, _CHUNK)
    i_idx3 = I_ids.reshape(_NW, n_chunks, _CHUNK)
    U_emb, I_emb = _sc_gather(user_table, item_table, u_idx3, i_idx3, B, D)
    R = _tc_mlp(U_emb, I_emb, W0, b0, W1, b1, W2, b2, W3, b3)
    return (U_emb, I_emb, R)
